# Initial kernel scaffold; baseline (speedup 1.0000x reference)
import jax
import jax.numpy as jnp
from jax.experimental import pallas as pl


def kernel(x_atom, x_aa, ei_aa_atom, ei_atom_aa, ei_bond, ea_aa_atom, ea_atom_aa, ea_bond, w1_a2t, b1_a2t, w2_a2t, b2_a2t, w1_t2a, b1_t2a, w2_t2a, b2_t2a, w1_t2t, b1_t2t, w2_t2t, b2_t2t, root_a2t, bias_a2t, root_t2a, bias_t2a, root_t2t, bias_t2t, proj_atom_w, proj_atom_b, proj_aa_w, proj_aa_b, cls1_w, cls1_b, cls2_w, cls2_b):
    raise NotImplementedError("write your pallas kernel here")


# speedup vs baseline: 3.1318x; 3.1318x over previous
import functools

import jax
import jax.numpy as jnp
from jax import lax
from jax.experimental import pallas as pl
from jax.experimental.pallas import tpu as pltpu
from jax.experimental.pallas import tpu_sc as plsc

N_ATOM = 50000; N_AA = 5000
E_AA_ATOM = 50000; E_ATOM_AA = 50000; E_BOND = 100000
D_ATOM = 16; D_AA = 32; D_EDGE = 16; DIM = 64

NC, NS = 2, 16
NW = NC * NS
CH = 128

H1 = 50176; H2 = 50176; H3 = 5120
OFF1, OFF2, OFF3 = 0, H1, H1 + H2
HTOT = H1 + H2 + H3
HSL = HTOT // NS

EH_TOT = E_AA_ATOM + E_BOND + E_ATOM_AA
KH = -(-EH_TOT // (NW * CH))
EH_PAD = NW * KH * CH

KG1 = -(-E_AA_ATOM // (NW * CH)); EP1 = NW * KG1 * CH
KG2 = -(-E_BOND // (NW * CH));    EP2 = NW * KG2 * CH
KG3 = -(-E_ATOM_AA // (NW * CH)); EP3 = NW * KG3 * CH

_mesh = plsc.VectorSubcoreMesh(core_axis_name="c", subcore_axis_name="s",
                               num_cores=NC, num_subcores=NS)


@functools.partial(
    pl.kernel, mesh=_mesh,
    out_type=jax.ShapeDtypeStruct((NC * HTOT,), jnp.float32),
    scratch_types=[
        pltpu.VMEM_SHARED((HTOT,), jnp.float32),
        pltpu.VMEM((KH, CH), jnp.int32),
        pltpu.VMEM((KH, CH), jnp.float32),
        pltpu.VMEM((HSL,), jnp.float32),
    ],
)
def _sc_histogram(idx3, val3, out, hist_sp, idx_v, val_v, stage_v):
    cid = lax.axis_index("c")
    sid = lax.axis_index("s")
    wid = sid * NC + cid

    def zbody(i, carry):
        stage_v[pl.ds(i * 16, 16)] = jnp.zeros((16,), jnp.float32)
        return carry

    lax.fori_loop(0, HSL // 16, zbody, 0)
    pltpu.sync_copy(stage_v, hist_sp.at[pl.ds(sid * HSL, HSL)])
    plsc.subcore_barrier()
    pltpu.sync_copy(idx3.at[wid], idx_v)
    pltpu.sync_copy(val3.at[wid], val_v)

    def body(j, carry):
        pltpu.sync_copy(val_v.at[j], hist_sp.at[idx_v.at[j]], add=True)
        return carry

    lax.fori_loop(0, KH, body, 0)
    plsc.subcore_barrier()
    pltpu.sync_copy(hist_sp.at[pl.ds(sid * HSL, HSL)], stage_v)
    pltpu.sync_copy(stage_v, out.at[pl.ds(cid * HTOT + sid * HSL, HSL)])


@functools.partial(
    pl.kernel, mesh=_mesh,
    compiler_params=pltpu.CompilerParams(use_tc_tiling_on_sc=False),
    out_type=(
        jax.ShapeDtypeStruct((EP1, D_AA), jnp.float32),
        jax.ShapeDtypeStruct((EP2, D_ATOM), jnp.float32),
        jax.ShapeDtypeStruct((EP3, D_ATOM), jnp.float32),
        jax.ShapeDtypeStruct((EP1,), jnp.float32),
        jax.ShapeDtypeStruct((EP2,), jnp.float32),
        jax.ShapeDtypeStruct((EP3,), jnp.float32),
    ),
    scratch_types=[
        pltpu.VMEM((KG1, CH), jnp.int32), pltpu.VMEM((KG1, CH), jnp.int32),
        pltpu.VMEM((KG2, CH), jnp.int32), pltpu.VMEM((KG2, CH), jnp.int32),
        pltpu.VMEM((KG3, CH), jnp.int32), pltpu.VMEM((KG3, CH), jnp.int32),
        pltpu.VMEM((CH, D_AA), jnp.float32),
        pltpu.VMEM((CH, D_ATOM), jnp.float32),
        pltpu.VMEM((CH,), jnp.float32),
    ],
)
def _sc_gather(src1, dst1, src2, dst2, src3, dst3, x_atom, x_aa, cnt,
               xg1, xg2, xg3, c1, c2, c3,
               s1_v, d1_v, s2_v, d2_v, s3_v, d3_v, row32_v, row16_v, cnt_v):
    cid = lax.axis_index("c")
    sid = lax.axis_index("s")
    wid = sid * NC + cid
    pltpu.sync_copy(src1.at[wid], s1_v)
    pltpu.sync_copy(dst1.at[wid], d1_v)
    pltpu.sync_copy(src2.at[wid], s2_v)
    pltpu.sync_copy(dst2.at[wid], d2_v)
    pltpu.sync_copy(src3.at[wid], s3_v)
    pltpu.sync_copy(dst3.at[wid], d3_v)

    def rel(kg, table, s_v, d_v, row_v, xg_out, c_out):
        base0 = wid * kg * CH

        def body(j, carry):
            base = base0 + j * CH
            pltpu.sync_copy(table.at[s_v.at[j]], row_v)
            pltpu.sync_copy(row_v, xg_out.at[pl.ds(base, CH)])
            pltpu.sync_copy(cnt.at[d_v.at[j]], cnt_v)
            pltpu.sync_copy(cnt_v, c_out.at[pl.ds(base, CH)])
            return carry

        lax.fori_loop(0, kg, body, 0)

    rel(KG1, x_aa, s1_v, d1_v, row32_v, xg1, c1)
    rel(KG2, x_atom, s2_v, d2_v, row16_v, xg2, c2)
    rel(KG3, x_atom, s3_v, d3_v, row16_v, xg3, c3)


def _edge_reduce_kernel(ea_ref, xg_ref, c_ref, w1t_ref, b1_ref, a_ref):
    b = pl.program_id(0)
    c = c_ref[...]
    w = jnp.where(c > 0.0, 1.0 / jnp.maximum(c, 1.0), 0.0)
    h = jax.nn.relu(
        jnp.dot(ea_ref[...], w1t_ref[...],
                preferred_element_type=jnp.float32) + b1_ref[...])
    hwa = jnp.concatenate([h * w, w], axis=1)
    contrib = lax.dot_general(
        xg_ref[...], hwa, (((0,), (0,)), ((), ())),
        preferred_element_type=jnp.float32)

    @pl.when(b == 0)
    def _():
        a_ref[...] = jnp.zeros_like(a_ref)

    a_ref[...] += contrib


def _edge_reduce(ea_pad, xg, cdst, w1t, b1, in_l, blk):
    e_pad = ea_pad.shape[0]
    nb = e_pad // blk
    return pl.pallas_call(
        _edge_reduce_kernel,
        grid=(nb,),
        in_specs=[
            pl.BlockSpec((blk, D_EDGE), lambda b: (b, 0)),
            pl.BlockSpec((blk, in_l), lambda b: (b, 0)),
            pl.BlockSpec((blk, 1), lambda b: (b, 0)),
            pl.BlockSpec((D_EDGE, DIM), lambda b: (0, 0)),
            pl.BlockSpec((1, DIM), lambda b: (0, 0)),
        ],
        out_specs=pl.BlockSpec((in_l, DIM + 1), lambda b: (0, 0)),
        out_shape=jax.ShapeDtypeStruct((in_l, DIM + 1), jnp.float32),
    )(ea_pad, xg, cdst, w1t, b1)


def _colsum_kernel(x_ref, o_ref):
    @pl.when(pl.program_id(0) == 0)
    def _():
        o_ref[...] = jnp.zeros_like(o_ref)

    o_ref[...] += jnp.sum(x_ref[...], axis=0, keepdims=True)


def _colsum(x, blk):
    n, d = x.shape
    return pl.pallas_call(
        _colsum_kernel,
        grid=(n // blk,),
        in_specs=[pl.BlockSpec((blk, d), lambda b: (b, 0))],
        out_specs=pl.BlockSpec((1, d), lambda b: (0, 0)),
        out_shape=jax.ShapeDtypeStruct((1, d), jnp.float32),
    )(x)


def _head_kernel(a1_ref, a2_ref, a3_ref, xsa_ref, xsaa_ref,
                 q1_ref, q2_ref, q3_ref, b2r1_ref, b2r2_ref, b2r3_ref,
                 r1t_ref, r2t_ref, r3t_ref,
                 pat_ref, pab_ref, paat_ref, paab_ref,
                 c1t_ref, c1b_ref, c2t_ref, c2b_ref, out_ref):
    def contract(a_ref, q_ref, b2r_ref, in_l, out_c):
        acc = jnp.zeros((1, out_c), dtype=jnp.float32)
        for i in range(in_l):
            mi = a_ref[i:i + 1, :DIM]
            qi = q_ref[i * DIM:(i + 1) * DIM, :]
            acc += jnp.dot(mi, qi, preferred_element_type=jnp.float32)
            acc += a_ref[i:i + 1, DIM:DIM + 1] * b2r_ref[i:i + 1, :]
        return acc

    t1 = contract(a1_ref, q1_ref, b2r1_ref, D_AA, D_ATOM) / N_ATOM
    t2 = contract(a2_ref, q2_ref, b2r2_ref, D_ATOM, D_ATOM) / N_ATOM
    t3 = contract(a3_ref, q3_ref, b2r3_ref, D_ATOM, D_AA) / N_AA

    xm_atom = xsa_ref[...] / N_ATOM
    xm_aa = xsaa_ref[...] / N_AA
    mo1 = t1 + jnp.dot(xm_atom, r1t_ref[...], preferred_element_type=jnp.float32)
    mo2 = t2 + jnp.dot(xm_atom, r2t_ref[...], preferred_element_type=jnp.float32)
    atom_mean = (mo1 + mo2) * 0.5
    aa_mean = t3 + jnp.dot(xm_aa, r3t_ref[...], preferred_element_type=jnp.float32)

    atom_p = jnp.dot(atom_mean, pat_ref[...],
                     preferred_element_type=jnp.float32) + pab_ref[...]
    aa_p = jnp.dot(aa_mean, paat_ref[...],
                   preferred_element_type=jnp.float32) + paab_ref[...]
    pooled = jnp.concatenate([atom_p, aa_p], axis=1)
    hh = jax.nn.relu(jnp.dot(pooled, c1t_ref[...],
                             preferred_element_type=jnp.float32) + c1b_ref[...])
    out_ref[...] = jnp.dot(hh, c2t_ref[...],
                           preferred_element_type=jnp.float32) + c2b_ref[...]


def _head(*args):
    return pl.pallas_call(
        _head_kernel,
        out_shape=jax.ShapeDtypeStruct((1, 1), jnp.float32),
    )(*args)


def _pad_idx(a, n_pad, fill):
    return jnp.concatenate(
        [a, jnp.full((n_pad - a.shape[0],), fill, jnp.int32)]).reshape(NW, -1, CH)


def kernel(x_atom, x_aa, ei_aa_atom, ei_atom_aa, ei_bond, ea_aa_atom,
           ea_atom_aa, ea_bond, w1_a2t, b1_a2t, w2_a2t, b2_a2t, w1_t2a, b1_t2a,
           w2_t2a, b2_t2a, w1_t2t, b1_t2t, w2_t2t, b2_t2t, root_a2t, bias_a2t,
           root_t2a, bias_t2a, root_t2t, bias_t2t, proj_atom_w, proj_atom_b,
           proj_aa_w, proj_aa_b, cls1_w, cls1_b, cls2_w, cls2_b):
    hidx = jnp.concatenate([ei_aa_atom[1], ei_bond[1] + OFF2,
                            ei_atom_aa[1] + OFF3])
    hidx3 = _pad_idx(hidx, EH_PAD, 0)
    hval3 = jnp.concatenate(
        [jnp.ones((EH_TOT,), jnp.float32),
         jnp.zeros((EH_PAD - EH_TOT,), jnp.float32)]).reshape(NW, KH, CH)

    hist_pair = _sc_histogram(hidx3, hval3).reshape(NC, HTOT)
    cnt = hist_pair[0] + hist_pair[1]

    zslot = N_ATOM + 100
    s1 = _pad_idx(ei_aa_atom[0], EP1, 0)
    d1 = _pad_idx(ei_aa_atom[1], EP1, zslot)
    s2 = _pad_idx(ei_bond[0], EP2, 0)
    d2 = _pad_idx(ei_bond[1] + OFF2, EP2, zslot)
    s3 = _pad_idx(ei_atom_aa[0], EP3, 0)
    d3 = _pad_idx(ei_atom_aa[1] + OFF3, EP3, zslot)

    xg1, xg2, xg3, c1, c2, c3 = _sc_gather(
        s1, d1, s2, d2, s3, d3, x_atom, x_aa, cnt)

    blk = 4096
    ea1 = jnp.pad(ea_aa_atom, ((0, EP1 - E_AA_ATOM), (0, 0)))
    ea2 = jnp.pad(ea_bond, ((0, EP2 - E_BOND), (0, 0)))
    ea3 = jnp.pad(ea_atom_aa, ((0, EP3 - E_ATOM_AA), (0, 0)))
    a1 = _edge_reduce(ea1, xg1, c1.reshape(EP1, 1), w1_a2t.T,
                      b1_a2t.reshape(1, DIM), D_AA, blk)
    a2 = _edge_reduce(ea2, xg2, c2.reshape(EP2, 1), w1_t2t.T,
                      b1_t2t.reshape(1, DIM), D_ATOM, blk)
    a3 = _edge_reduce(ea3, xg3, c3.reshape(EP3, 1), w1_t2a.T,
                      b1_t2a.reshape(1, DIM), D_ATOM, blk)

    xs_atom = _colsum(x_atom, 2000)
    xs_aa = _colsum(x_aa, 5000)

    q1 = w2_a2t.reshape(D_AA, D_ATOM, DIM).transpose(0, 2, 1).reshape(
        D_AA * DIM, D_ATOM)
    q2 = w2_t2t.reshape(D_ATOM, D_ATOM, DIM).transpose(0, 2, 1).reshape(
        D_ATOM * DIM, D_ATOM)
    q3 = w2_t2a.reshape(D_ATOM, D_AA, DIM).transpose(0, 2, 1).reshape(
        D_ATOM * DIM, D_AA)

    out = _head(a1, a2, a3, xs_atom, xs_aa, q1, q2, q3,
                b2_a2t.reshape(D_AA, D_ATOM), b2_t2t.reshape(D_ATOM, D_ATOM),
                b2_t2a.reshape(D_ATOM, D_AA),
                root_a2t.T, root_t2t.T, root_t2a.T,
                proj_atom_w.T, proj_atom_b.reshape(1, DIM),
                proj_aa_w.T, proj_aa_b.reshape(1, DIM),
                cls1_w.T, cls1_b.reshape(1, DIM),
                cls2_w.T, cls2_b.reshape(1, 1))
    return out.reshape(-1)


# speedup vs baseline: 3.3130x; 1.0579x over previous
import functools

import jax
import jax.numpy as jnp
from jax import lax
from jax.experimental import pallas as pl
from jax.experimental.pallas import tpu as pltpu
from jax.experimental.pallas import tpu_sc as plsc

N_ATOM = 50000; N_AA = 5000
E_AA_ATOM = 50000; E_ATOM_AA = 50000; E_BOND = 100000
D_ATOM = 16; D_AA = 32; D_EDGE = 16; DIM = 64

NC, NS = 2, 16
NW = NC * NS
CH = 128

H1 = 50176; H2 = 50176; H3 = 5120
OFF1, OFF2, OFF3 = 0, H1, H1 + H2
HTOT = H1 + H2 + H3
HSL = HTOT // NS

EH_TOT = E_AA_ATOM + E_BOND + E_ATOM_AA
KH = -(-EH_TOT // (NW * CH))
EH_PAD = NW * KH * CH

KG1 = -(-E_AA_ATOM // (NW * CH)); EP1 = NW * KG1 * CH
KG2 = -(-E_BOND // (NW * CH));    EP2 = NW * KG2 * CH
KG3 = -(-E_ATOM_AA // (NW * CH)); EP3 = NW * KG3 * CH
P32 = 7
P16 = 10

_mesh = plsc.VectorSubcoreMesh(core_axis_name="c", subcore_axis_name="s",
                               num_cores=NC, num_subcores=NS)


@functools.partial(
    pl.kernel, mesh=_mesh,
    out_type=jax.ShapeDtypeStruct((NC * HTOT,), jnp.float32),
    scratch_types=[
        pltpu.VMEM_SHARED((HTOT,), jnp.float32),
        pltpu.VMEM((KH, CH), jnp.int32),
        pltpu.VMEM((KH, CH), jnp.float32),
        pltpu.VMEM((HSL,), jnp.float32),
    ],
)
def _sc_histogram(idx3, val3, out, hist_sp, idx_v, val_v, stage_v):
    cid = lax.axis_index("c")
    sid = lax.axis_index("s")
    wid = sid * NC + cid

    def zbody(i, carry):
        stage_v[pl.ds(i * 16, 16)] = jnp.zeros((16,), jnp.float32)
        return carry

    lax.fori_loop(0, HSL // 16, zbody, 0)
    pltpu.sync_copy(stage_v, hist_sp.at[pl.ds(sid * HSL, HSL)])
    plsc.subcore_barrier()
    pltpu.sync_copy(idx3.at[wid], idx_v)
    pltpu.sync_copy(val3.at[wid], val_v)

    def body(j, carry):
        pltpu.sync_copy(val_v.at[j], hist_sp.at[idx_v.at[j]], add=True)
        return carry

    lax.fori_loop(0, KH, body, 0)
    plsc.subcore_barrier()
    pltpu.sync_copy(hist_sp.at[pl.ds(sid * HSL, HSL)], stage_v)
    pltpu.sync_copy(stage_v, out.at[pl.ds(cid * HTOT + sid * HSL, HSL)])


@functools.partial(
    pl.kernel, mesh=_mesh,
    compiler_params=pltpu.CompilerParams(use_tc_tiling_on_sc=False),
    out_type=(
        jax.ShapeDtypeStruct((EP1, D_AA), jnp.float32),
        jax.ShapeDtypeStruct((EP2, D_ATOM), jnp.float32),
        jax.ShapeDtypeStruct((EP3, D_ATOM), jnp.float32),
        jax.ShapeDtypeStruct((EP1,), jnp.float32),
        jax.ShapeDtypeStruct((EP2,), jnp.float32),
        jax.ShapeDtypeStruct((EP3,), jnp.float32),
    ),
    scratch_types=[
        pltpu.VMEM((KG1, CH), jnp.int32), pltpu.VMEM((KG1, CH), jnp.int32),
        pltpu.VMEM((KG2, CH), jnp.int32), pltpu.VMEM((KG2, CH), jnp.int32),
        pltpu.VMEM((KG3, CH), jnp.int32), pltpu.VMEM((KG3, CH), jnp.int32),
        pltpu.VMEM((P32 * CH, D_AA), jnp.float32),
        pltpu.VMEM((P32 * CH, D_AA), jnp.float32),
        pltpu.VMEM((P16 * CH, D_ATOM), jnp.float32),
        pltpu.VMEM((P16 * CH, D_ATOM), jnp.float32),
        pltpu.VMEM((KG1 * CH,), jnp.float32),
        pltpu.VMEM((KG2 * CH,), jnp.float32),
        pltpu.VMEM((KG3 * CH,), jnp.float32),
    ] + [pltpu.SemaphoreType.DMA] * 11,
)
def _sc_gather(src1, dst1, src2, dst2, src3, dst3, x_atom, x_aa, cnt,
               xg1, xg2, xg3, c1, c2, c3,
               s1_v, d1_v, s2_v, d2_v, s3_v, d3_v,
               rowa32, rowb32, rowa16, rowb16, c1_v, c2_v, c3_v,
               semi, semg0, semg1, semg2, semg3, semw0, semw1, semw2, semw3,
               semcg, semcw):
    cid = lax.axis_index("c")
    sid = lax.axis_index("s")
    wid = sid * NC + cid

    loads = [pltpu.async_copy(src, dst, semi) for src, dst in
             ((src1.at[wid], s1_v), (dst1.at[wid], d1_v),
              (src2.at[wid], s2_v), (dst2.at[wid], d2_v),
              (src3.at[wid], s3_v), (dst3.at[wid], d3_v))]
    for ld in loads:
        ld.wait()

    slots32 = [[rowa32, semg0, semw0, None], [rowb32, semg1, semw1, None]]
    slots16 = [[rowa16, semg2, semw2, None], [rowb16, semg3, semw3, None]]
    cnt_writes = []

    def rel(kg, pch, table, s_v, d_v, slots, c_v, xg_out, c_out):
        base0 = wid * kg * CH
        phases = []
        j0 = 0
        while j0 < kg:
            phases.append((j0, min(pch, kg - j0)))
            j0 += pch
        gdescs = [None] * len(phases)
        cdescs = []

        def write_phase(p):
            j0, n = phases[p]
            slot = slots[p % 2]
            for dsc in gdescs[p]:
                dsc.wait()
            slot[3] = pltpu.async_copy(
                slot[0].at[pl.ds(0, n * CH)],
                xg_out.at[pl.ds(base0 + j0 * CH, n * CH)], slot[2])

        for p, (j0, n) in enumerate(phases):
            slot = slots[p % 2]
            if slot[3] is not None:
                slot[3].wait()
                slot[3] = None
            gdescs[p] = [
                pltpu.async_copy(table.at[s_v.at[j0 + j]],
                                 slot[0].at[pl.ds(j * CH, CH)], slot[1])
                for j in range(n)]
            cdescs.extend(
                pltpu.async_copy(cnt.at[d_v.at[j0 + j]],
                                 c_v.at[pl.ds((j0 + j) * CH, CH)], semcg)
                for j in range(n))
            if p >= 1:
                write_phase(p - 1)
        write_phase(len(phases) - 1)
        for dsc in cdescs:
            dsc.wait()
        cnt_writes.append(pltpu.async_copy(
            c_v, c_out.at[pl.ds(wid * kg * CH, kg * CH)], semcw))

    rel(KG1, P32, x_aa, s1_v, d1_v, slots32, c1_v, xg1, c1)
    rel(KG2, P16, x_atom, s2_v, d2_v, slots16, c2_v, xg2, c2)
    rel(KG3, P16, x_atom, s3_v, d3_v, slots16, c3_v, xg3, c3)

    for slot in slots32 + slots16:
        if slot[3] is not None:
            slot[3].wait()
    for dsc in cnt_writes:
        dsc.wait()


def _edge_reduce_kernel(ea_ref, xg_ref, c_ref, w1t_ref, b1_ref, a_ref):
    b = pl.program_id(0)
    c = c_ref[...]
    w = jnp.where(c > 0.0, 1.0 / jnp.maximum(c, 1.0), 0.0)
    h = jax.nn.relu(
        jnp.dot(ea_ref[...], w1t_ref[...],
                preferred_element_type=jnp.float32) + b1_ref[...])
    hwa = jnp.concatenate([h * w, w], axis=1)
    contrib = lax.dot_general(
        xg_ref[...], hwa, (((0,), (0,)), ((), ())),
        preferred_element_type=jnp.float32)

    @pl.when(b == 0)
    def _():
        a_ref[...] = jnp.zeros_like(a_ref)

    a_ref[...] += contrib


def _edge_reduce(ea_pad, xg, cdst, w1t, b1, in_l, blk):
    e_pad = ea_pad.shape[0]
    nb = e_pad // blk
    return pl.pallas_call(
        _edge_reduce_kernel,
        grid=(nb,),
        in_specs=[
            pl.BlockSpec((blk, D_EDGE), lambda b: (b, 0)),
            pl.BlockSpec((blk, in_l), lambda b: (b, 0)),
            pl.BlockSpec((blk, 1), lambda b: (b, 0)),
            pl.BlockSpec((D_EDGE, DIM), lambda b: (0, 0)),
            pl.BlockSpec((1, DIM), lambda b: (0, 0)),
        ],
        out_specs=pl.BlockSpec((in_l, DIM + 1), lambda b: (0, 0)),
        out_shape=jax.ShapeDtypeStruct((in_l, DIM + 1), jnp.float32),
    )(ea_pad, xg, cdst, w1t, b1)


def _colsum_kernel(x_ref, o_ref):
    @pl.when(pl.program_id(0) == 0)
    def _():
        o_ref[...] = jnp.zeros_like(o_ref)

    o_ref[...] += jnp.sum(x_ref[...], axis=0, keepdims=True)


def _colsum(x, blk):
    n, d = x.shape
    return pl.pallas_call(
        _colsum_kernel,
        grid=(n // blk,),
        in_specs=[pl.BlockSpec((blk, d), lambda b: (b, 0))],
        out_specs=pl.BlockSpec((1, d), lambda b: (0, 0)),
        out_shape=jax.ShapeDtypeStruct((1, d), jnp.float32),
    )(x)


def _head_kernel(a1_ref, a2_ref, a3_ref, xsa_ref, xsaa_ref,
                 q1_ref, q2_ref, q3_ref, b2r1_ref, b2r2_ref, b2r3_ref,
                 r1t_ref, r2t_ref, r3t_ref,
                 pat_ref, pab_ref, paat_ref, paab_ref,
                 c1t_ref, c1b_ref, c2t_ref, c2b_ref, out_ref):
    def contract(a_ref, q_ref, b2r_ref, in_l, out_c):
        acc = jnp.zeros((1, out_c), dtype=jnp.float32)
        for i in range(in_l):
            mi = a_ref[i:i + 1, :DIM]
            qi = q_ref[i * DIM:(i + 1) * DIM, :]
            acc += jnp.dot(mi, qi, preferred_element_type=jnp.float32)
            acc += a_ref[i:i + 1, DIM:DIM + 1] * b2r_ref[i:i + 1, :]
        return acc

    t1 = contract(a1_ref, q1_ref, b2r1_ref, D_AA, D_ATOM) / N_ATOM
    t2 = contract(a2_ref, q2_ref, b2r2_ref, D_ATOM, D_ATOM) / N_ATOM
    t3 = contract(a3_ref, q3_ref, b2r3_ref, D_ATOM, D_AA) / N_AA

    xm_atom = xsa_ref[...] / N_ATOM
    xm_aa = xsaa_ref[...] / N_AA
    mo1 = t1 + jnp.dot(xm_atom, r1t_ref[...], preferred_element_type=jnp.float32)
    mo2 = t2 + jnp.dot(xm_atom, r2t_ref[...], preferred_element_type=jnp.float32)
    atom_mean = (mo1 + mo2) * 0.5
    aa_mean = t3 + jnp.dot(xm_aa, r3t_ref[...], preferred_element_type=jnp.float32)

    atom_p = jnp.dot(atom_mean, pat_ref[...],
                     preferred_element_type=jnp.float32) + pab_ref[...]
    aa_p = jnp.dot(aa_mean, paat_ref[...],
                   preferred_element_type=jnp.float32) + paab_ref[...]
    pooled = jnp.concatenate([atom_p, aa_p], axis=1)
    hh = jax.nn.relu(jnp.dot(pooled, c1t_ref[...],
                             preferred_element_type=jnp.float32) + c1b_ref[...])
    out_ref[...] = jnp.dot(hh, c2t_ref[...],
                           preferred_element_type=jnp.float32) + c2b_ref[...]


def _head(*args):
    return pl.pallas_call(
        _head_kernel,
        out_shape=jax.ShapeDtypeStruct((1, 1), jnp.float32),
    )(*args)


def _pad_idx(a, n_pad, fill):
    return jnp.concatenate(
        [a, jnp.full((n_pad - a.shape[0],), fill, jnp.int32)]).reshape(NW, -1, CH)


def kernel(x_atom, x_aa, ei_aa_atom, ei_atom_aa, ei_bond, ea_aa_atom,
           ea_atom_aa, ea_bond, w1_a2t, b1_a2t, w2_a2t, b2_a2t, w1_t2a, b1_t2a,
           w2_t2a, b2_t2a, w1_t2t, b1_t2t, w2_t2t, b2_t2t, root_a2t, bias_a2t,
           root_t2a, bias_t2a, root_t2t, bias_t2t, proj_atom_w, proj_atom_b,
           proj_aa_w, proj_aa_b, cls1_w, cls1_b, cls2_w, cls2_b):
    hidx = jnp.concatenate([ei_aa_atom[1], ei_bond[1] + OFF2,
                            ei_atom_aa[1] + OFF3])
    hidx3 = _pad_idx(hidx, EH_PAD, 0)
    hval3 = jnp.concatenate(
        [jnp.ones((EH_TOT,), jnp.float32),
         jnp.zeros((EH_PAD - EH_TOT,), jnp.float32)]).reshape(NW, KH, CH)

    hist_pair = _sc_histogram(hidx3, hval3).reshape(NC, HTOT)
    cnt = hist_pair[0] + hist_pair[1]

    zslot = N_ATOM + 100
    s1 = _pad_idx(ei_aa_atom[0], EP1, 0)
    d1 = _pad_idx(ei_aa_atom[1], EP1, zslot)
    s2 = _pad_idx(ei_bond[0], EP2, 0)
    d2 = _pad_idx(ei_bond[1] + OFF2, EP2, zslot)
    s3 = _pad_idx(ei_atom_aa[0], EP3, 0)
    d3 = _pad_idx(ei_atom_aa[1] + OFF3, EP3, zslot)

    xg1, xg2, xg3, c1, c2, c3 = _sc_gather(
        s1, d1, s2, d2, s3, d3, x_atom, x_aa, cnt)

    blk = 4096
    ea1 = jnp.pad(ea_aa_atom, ((0, EP1 - E_AA_ATOM), (0, 0)))
    ea2 = jnp.pad(ea_bond, ((0, EP2 - E_BOND), (0, 0)))
    ea3 = jnp.pad(ea_atom_aa, ((0, EP3 - E_ATOM_AA), (0, 0)))
    a1 = _edge_reduce(ea1, xg1, c1.reshape(EP1, 1), w1_a2t.T,
                      b1_a2t.reshape(1, DIM), D_AA, blk)
    a2 = _edge_reduce(ea2, xg2, c2.reshape(EP2, 1), w1_t2t.T,
                      b1_t2t.reshape(1, DIM), D_ATOM, blk)
    a3 = _edge_reduce(ea3, xg3, c3.reshape(EP3, 1), w1_t2a.T,
                      b1_t2a.reshape(1, DIM), D_ATOM, blk)

    xs_atom = _colsum(x_atom, 2000)
    xs_aa = _colsum(x_aa, 5000)

    q1 = w2_a2t.reshape(D_AA, D_ATOM, DIM).transpose(0, 2, 1).reshape(
        D_AA * DIM, D_ATOM)
    q2 = w2_t2t.reshape(D_ATOM, D_ATOM, DIM).transpose(0, 2, 1).reshape(
        D_ATOM * DIM, D_ATOM)
    q3 = w2_t2a.reshape(D_ATOM, D_AA, DIM).transpose(0, 2, 1).reshape(
        D_ATOM * DIM, D_AA)

    out = _head(a1, a2, a3, xs_atom, xs_aa, q1, q2, q3,
                b2_a2t.reshape(D_AA, D_ATOM), b2_t2t.reshape(D_ATOM, D_ATOM),
                b2_t2a.reshape(D_ATOM, D_AA),
                root_a2t.T, root_t2t.T, root_t2a.T,
                proj_atom_w.T, proj_atom_b.reshape(1, DIM),
                proj_aa_w.T, proj_aa_b.reshape(1, DIM),
                cls1_w.T, cls1_b.reshape(1, DIM),
                cls2_w.T, cls2_b.reshape(1, 1))
    return out.reshape(-1)
